# baseline (speedup 1.0000x reference)
import functools

import jax
import jax.numpy as jnp
from jax import lax
from jax.experimental import pallas as pl
from jax.experimental.pallas import tpu as pltpu
from jax.experimental.pallas import tpu_sc as plsc

B = 16384
NW = 32
BPW = B // NW
CHUNK = 16
NCHUNK = BPW // CHUNK


@functools.cache
def _sc_gather_fn(nb):
    bpw = nb // NW
    nchunk = bpw // CHUNK
    mesh = plsc.VectorSubcoreMesh(core_axis_name="c", subcore_axis_name="s")

    @functools.partial(
        pl.kernel,
        out_type=jax.ShapeDtypeStruct((nb, 64), jnp.float32),
        mesh=mesh,
        compiler_params=pltpu.CompilerParams(needs_layout_passes=False),
        scratch_types=[
            pltpu.VMEM((bpw,), jnp.int32),
            pltpu.VMEM((CHUNK, 8, 64), jnp.float32),
            pltpu.VMEM((CHUNK, 8, 64), jnp.float32),
            pltpu.VMEM((bpw, 64), jnp.float32),
            pltpu.SemaphoreType.DMA,
            pltpu.SemaphoreType.DMA,
        ],
    )
    def _sc_gather(tbl3, iid, i_out,
                   iidx_v, tile0, tile1, iout_v, sem0, sem1):
        wid = lax.axis_index("s") * 2 + lax.axis_index("c")
        base = wid * bpw
        pltpu.sync_copy(iid.at[pl.ds(base, bpw)], iidx_v)

        sems = (sem0, sem1)
        bufs = (tile0, tile1)
        pend = [None, None]
        iota = lax.iota(jnp.int32, 16)

        def fire(c):
            buf = bufs[c % 2]
            ids = iidx_v[pl.ds(c * CHUNK, CHUNK)]
            cps = []
            for k in range(CHUNK):
                g = lax.div(ids[k], 8)
                cps.append(pltpu.async_copy(
                    tbl3.at[pl.ds(g, 1)], buf.at[pl.ds(k, 1)],
                    sems[c % 2]))
            return cps

        def extract(c):
            buf = bufs[c % 2]

            def body(k, carry):
                j = c * CHUNK + k
                s16 = plsc.load_gather(iidx_v, [jnp.full((16,), j, jnp.int32)])
                r16 = lax.rem(s16, 8)
                k16 = jnp.full((16,), k, jnp.int32)
                for m in range(4):
                    vals = plsc.load_gather(buf, [k16, r16, iota + 16 * m])
                    iout_v[j, pl.ds(16 * m, 16)] = vals
                return carry

            lax.fori_loop(0, CHUNK, body, 0)

        pend[0] = fire(0)
        for c in range(nchunk):
            if c + 1 < nchunk:
                pend[(c + 1) % 2] = fire(c + 1)
            for cp in pend[c % 2]:
                cp.wait()
            extract(c)

        pltpu.sync_copy(iout_v, i_out.at[pl.ds(base, bpw)])

    return _sc_gather


@functools.cache
def _sc_cat_gather_fn():
    mesh = plsc.VectorSubcoreMesh(core_axis_name="c", subcore_axis_name="s")

    @functools.partial(
        pl.kernel,
        out_type=jax.ShapeDtypeStruct((B, 32), jnp.float32),
        mesh=mesh,
        compiler_params=pltpu.CompilerParams(use_tc_tiling_on_sc=False),
        scratch_types=[
            pltpu.VMEM((BPW,), jnp.int32),
            pltpu.VMEM((BPW, 32), jnp.float32),
            pltpu.SemaphoreType.DMA,
        ],
    )
    def _sc_cat(cat_tbl, cid, c_out, cidx_v, crows_v, sem):
        wid = lax.axis_index("s") * 2 + lax.axis_index("c")
        base = wid * BPW
        pltpu.sync_copy(cid.at[pl.ds(base, BPW)], cidx_v)
        pltpu.async_copy(cat_tbl.at[cidx_v], crows_v, sem).wait()
        pltpu.sync_copy(crows_v, c_out.at[pl.ds(base, BPW)])

    return _sc_cat


def _mlp_body(i_ref, c_ref, p_ref, wp_ref, bp_ref, w1a_ref, w1b_ref,
              w1c_ref, b1_ref, w2_ref, b2_ref, g_ref, bt_ref, o_ref):
    p = p_ref[...] * wp_ref[...] + bp_ref[...]
    h = jnp.dot(i_ref[...], w1a_ref[...], preferred_element_type=jnp.float32)
    h = h + jnp.dot(c_ref[...], w1b_ref[...], preferred_element_type=jnp.float32)
    h = h + jnp.dot(p, w1c_ref[...], preferred_element_type=jnp.float32)
    h = jnp.maximum(h + b1_ref[...], 0.0)
    o = jnp.dot(h, w2_ref[...], preferred_element_type=jnp.float32) + b2_ref[...]
    mu = jnp.mean(o, axis=-1, keepdims=True)
    d = o - mu
    var = jnp.mean(d * d, axis=-1, keepdims=True)
    o_ref[...] = d * lax.rsqrt(var + 1e-5) * g_ref[...] + bt_ref[...]


_MLP_BLOCK = 1024


def _mlp_call(i, c, p2, Wp, bp2, W1a, W1b, W1c, b12, W2, b22, g2, bt2):
    nb = i.shape[0]
    nblk = nb // _MLP_BLOCK
    full = lambda shape: pl.BlockSpec(shape, lambda ib: (0, 0))
    return pl.pallas_call(
        _mlp_body,
        grid=(nblk,),
        in_specs=[
            pl.BlockSpec((_MLP_BLOCK, 64), lambda ib: (ib, 0)),
            pl.BlockSpec((_MLP_BLOCK, 32), lambda ib: (ib, 0)),
            pl.BlockSpec((_MLP_BLOCK, 1), lambda ib: (ib, 0)),
            full((1, 32)),
            full((1, 32)),
            full((64, 256)),
            full((32, 256)),
            full((32, 256)),
            full((1, 256)),
            full((256, 128)),
            full((1, 128)),
            full((1, 128)),
            full((1, 128)),
        ],
        out_specs=pl.BlockSpec((_MLP_BLOCK, 128), lambda ib: (ib, 0)),
        out_shape=jax.ShapeDtypeStruct((nb, 128), jnp.float32),
    )(i, c, p2, Wp, bp2, W1a, W1b, W1c, b12, W2, b22, g2, bt2)


def kernel(item_id, category_id, price, item_table, cat_table,
           Wp, bp, W1, b1, W2, b2, gamma, beta):
    tbl3 = item_table.reshape(125000, 8, 64)
    iid = item_id.astype(jnp.int32)
    c = _sc_cat_gather_fn()(cat_table, category_id.astype(jnp.int32))
    half = B // 2
    gf = _sc_gather_fn(half)
    outs = []
    for lo in (0, half):
        i_h = gf(tbl3, lax.dynamic_slice_in_dim(iid, lo, half))
        outs.append(_mlp_call(
            i_h, lax.dynamic_slice_in_dim(c, lo, half),
            price[lo:lo + half, None], Wp, bp[None, :],
            W1[:64], W1[64:96], W1[96:128], b1[None, :],
            W2, b2[None, :], gamma[None, :], beta[None, :]))
    return jnp.concatenate(outs, axis=0)


# speedup vs baseline: 1.0533x; 1.0533x over previous
import functools

import jax
import jax.numpy as jnp
from jax import lax
from jax.experimental import pallas as pl
from jax.experimental.pallas import tpu as pltpu
from jax.experimental.pallas import tpu_sc as plsc

B = 16384
NW = 32
BPW = B // NW
CHUNK = 32
NCHUNK = BPW // CHUNK


@functools.cache
def _sc_gather_fn():
    mesh = plsc.VectorSubcoreMesh(core_axis_name="c", subcore_axis_name="s")

    @functools.partial(
        pl.kernel,
        out_type=jax.ShapeDtypeStruct((B, 64), jnp.float32),
        mesh=mesh,
        compiler_params=pltpu.CompilerParams(needs_layout_passes=False),
        scratch_types=[
            pltpu.VMEM((BPW,), jnp.int32),
            pltpu.VMEM((CHUNK, 8, 64), jnp.float32),
            pltpu.VMEM((CHUNK, 8, 64), jnp.float32),
            pltpu.VMEM((CHUNK, 64), jnp.float32),
            pltpu.VMEM((CHUNK, 64), jnp.float32),
            pltpu.SemaphoreType.DMA,
            pltpu.SemaphoreType.DMA,
            pltpu.SemaphoreType.DMA,
        ],
    )
    def _sc_gather(tbl3, iid, i_out,
                   iidx_v, tile0, tile1, out0, out1, sem0, sem1, osem):
        wid = lax.axis_index("s") * 2 + lax.axis_index("c")
        base = wid * BPW
        pltpu.sync_copy(iid.at[pl.ds(base, BPW)], iidx_v)

        sems = (sem0, sem1)
        bufs = (tile0, tile1)
        pend = [None, None]
        iota = lax.iota(jnp.int32, 16)

        def fire(c):
            buf = bufs[c % 2]
            idparts = [iidx_v[pl.ds(c * CHUNK + 16 * q, 16)]
                       for q in range(CHUNK // 16)]
            cps = []
            for k in range(CHUNK):
                g = lax.div(idparts[k // 16][k % 16], 8)
                cps.append(pltpu.async_copy(
                    tbl3.at[pl.ds(g, 1)], buf.at[pl.ds(k, 1)],
                    sems[c % 2]))
            return cps

        obufs = (out0, out1)

        def extract(c):
            buf = bufs[c % 2]
            obuf = obufs[c % 2]

            def body(k, carry):
                j = c * CHUNK + k
                s16 = plsc.load_gather(iidx_v, [jnp.full((16,), j, jnp.int32)])
                r16 = lax.rem(s16, 8)
                k16 = jnp.full((16,), k, jnp.int32)
                for m in range(4):
                    vals = plsc.load_gather(buf, [k16, r16, iota + 16 * m])
                    obuf[k, pl.ds(16 * m, 16)] = vals
                return carry

            lax.fori_loop(0, CHUNK, body, 0)

        pend[0] = fire(0)
        pend_out = [None, None]
        for c in range(NCHUNK):
            if c + 1 < NCHUNK:
                pend[(c + 1) % 2] = fire(c + 1)
            for cp in pend[c % 2]:
                cp.wait()
            if pend_out[c % 2] is not None:
                pend_out[c % 2].wait()
            extract(c)
            pend_out[c % 2] = pltpu.async_copy(
                obufs[c % 2], i_out.at[pl.ds(base + c * CHUNK, CHUNK)], osem)
        for po in pend_out:
            if po is not None:
                po.wait()

    return _sc_gather


@functools.cache
def _sc_cat_gather_fn():
    mesh = plsc.VectorSubcoreMesh(core_axis_name="c", subcore_axis_name="s")

    @functools.partial(
        pl.kernel,
        out_type=jax.ShapeDtypeStruct((B, 32), jnp.float32),
        mesh=mesh,
        compiler_params=pltpu.CompilerParams(use_tc_tiling_on_sc=False),
        scratch_types=[
            pltpu.VMEM((BPW,), jnp.int32),
            pltpu.VMEM((BPW, 32), jnp.float32),
            pltpu.SemaphoreType.DMA,
        ],
    )
    def _sc_cat(cat_tbl, cid, c_out, cidx_v, crows_v, sem):
        wid = lax.axis_index("s") * 2 + lax.axis_index("c")
        base = wid * BPW
        pltpu.sync_copy(cid.at[pl.ds(base, BPW)], cidx_v)
        pltpu.async_copy(cat_tbl.at[cidx_v], crows_v, sem).wait()
        pltpu.sync_copy(crows_v, c_out.at[pl.ds(base, BPW)])

    return _sc_cat


def _mlp_body(i_ref, c_ref, p_ref, wp_ref, bp_ref, w1a_ref, w1b_ref,
              w1c_ref, b1_ref, w2_ref, b2_ref, g_ref, bt_ref, o_ref):
    p = p_ref[...] * wp_ref[...] + bp_ref[...]
    h = jnp.dot(i_ref[...], w1a_ref[...], preferred_element_type=jnp.float32)
    h = h + jnp.dot(c_ref[...], w1b_ref[...], preferred_element_type=jnp.float32)
    h = h + jnp.dot(p, w1c_ref[...], preferred_element_type=jnp.float32)
    h = jnp.maximum(h + b1_ref[...], 0.0)
    o = jnp.dot(h, w2_ref[...], preferred_element_type=jnp.float32) + b2_ref[...]
    mu = jnp.mean(o, axis=-1, keepdims=True)
    d = o - mu
    var = jnp.mean(d * d, axis=-1, keepdims=True)
    o_ref[...] = d * lax.rsqrt(var + 1e-5) * g_ref[...] + bt_ref[...]


_MLP_BLOCK = 1024


def _mlp_call(i, c, p2, Wp, bp2, W1a, W1b, W1c, b12, W2, b22, g2, bt2):
    nblk = B // _MLP_BLOCK
    full = lambda shape: pl.BlockSpec(shape, lambda ib: (0, 0))
    return pl.pallas_call(
        _mlp_body,
        grid=(nblk,),
        in_specs=[
            pl.BlockSpec((_MLP_BLOCK, 64), lambda ib: (ib, 0)),
            pl.BlockSpec((_MLP_BLOCK, 32), lambda ib: (ib, 0)),
            pl.BlockSpec((_MLP_BLOCK, 1), lambda ib: (ib, 0)),
            full((1, 32)),
            full((1, 32)),
            full((64, 256)),
            full((32, 256)),
            full((32, 256)),
            full((1, 256)),
            full((256, 128)),
            full((1, 128)),
            full((1, 128)),
            full((1, 128)),
        ],
        out_specs=pl.BlockSpec((_MLP_BLOCK, 128), lambda ib: (ib, 0)),
        out_shape=jax.ShapeDtypeStruct((B, 128), jnp.float32),
    )(i, c, p2, Wp, bp2, W1a, W1b, W1c, b12, W2, b22, g2, bt2)


def kernel(item_id, category_id, price, item_table, cat_table,
           Wp, bp, W1, b1, W2, b2, gamma, beta):
    i = _sc_gather_fn()(item_table.reshape(125000, 8, 64),
                        item_id.astype(jnp.int32))
    c = _sc_cat_gather_fn()(cat_table, category_id.astype(jnp.int32))
    return _mlp_call(
        i, c, price[:, None], Wp, bp[None, :],
        W1[:64], W1[64:96], W1[96:128], b1[None, :],
        W2, b2[None, :], gamma[None, :], beta[None, :])


# speedup vs baseline: 1.0704x; 1.0163x over previous
import functools

import jax
import jax.numpy as jnp
from jax import lax
from jax.experimental import pallas as pl
from jax.experimental.pallas import tpu as pltpu
from jax.experimental.pallas import tpu_sc as plsc

B = 16384
NW = 32
BPW = B // NW
CHUNK = 32
NCHUNK = BPW // CHUNK


@functools.cache
def _sc_gather_fn():
    mesh = plsc.VectorSubcoreMesh(core_axis_name="c", subcore_axis_name="s")

    @functools.partial(
        pl.kernel,
        out_type=jax.ShapeDtypeStruct((B, 64), jnp.float32),
        mesh=mesh,
        compiler_params=pltpu.CompilerParams(needs_layout_passes=False),
        scratch_types=[
            pltpu.VMEM((BPW,), jnp.int32),
            pltpu.VMEM((CHUNK, 8, 64), jnp.float32),
            pltpu.VMEM((CHUNK, 8, 64), jnp.float32),
            pltpu.VMEM((CHUNK, 64), jnp.float32),
            pltpu.VMEM((CHUNK, 64), jnp.float32),
            pltpu.SemaphoreType.DMA,
            pltpu.SemaphoreType.DMA,
            pltpu.SemaphoreType.DMA,
        ],
    )
    def _sc_gather(tbl3, iid, i_out,
                   iidx_v, tile0, tile1, out0, out1, sem0, sem1, osem):
        wid = lax.axis_index("s") * 2 + lax.axis_index("c")
        base = wid * BPW
        pltpu.sync_copy(iid.at[pl.ds(base, BPW)], iidx_v)

        sems = (sem0, sem1)
        bufs = (tile0, tile1)
        pend = [None, None]
        iota = lax.iota(jnp.int32, 16)

        def fire(c):
            buf = bufs[c % 2]
            idparts = [iidx_v[pl.ds(c * CHUNK + 16 * q, 16)]
                       for q in range(CHUNK // 16)]
            cps = []
            for k in range(CHUNK):
                g = lax.div(idparts[k // 16][k % 16], 8)
                cps.append(pltpu.async_copy(
                    tbl3.at[pl.ds(g, 1)], buf.at[pl.ds(k, 1)],
                    sems[c % 2]))
            return cps

        obufs = (out0, out1)

        def extract(c):
            buf = bufs[c % 2]
            obuf = obufs[c % 2]

            def body(k, carry):
                j = c * CHUNK + k
                s16 = plsc.load_gather(iidx_v, [jnp.full((16,), j, jnp.int32)])
                r16 = lax.rem(s16, 8)
                k16 = jnp.full((16,), k, jnp.int32)
                for m in range(4):
                    vals = plsc.load_gather(buf, [k16, r16, iota + 16 * m])
                    obuf[k, pl.ds(16 * m, 16)] = vals
                return carry

            lax.fori_loop(0, CHUNK, body, 0)

        pend[0] = fire(0)
        pend_out = [None, None]
        for c in range(NCHUNK):
            if c + 1 < NCHUNK:
                pend[(c + 1) % 2] = fire(c + 1)
            for cp in pend[c % 2]:
                cp.wait()
            if pend_out[c % 2] is not None:
                pend_out[c % 2].wait()
            extract(c)
            pend_out[c % 2] = pltpu.async_copy(
                obufs[c % 2], i_out.at[pl.ds(base + c * CHUNK, CHUNK)], osem)
        for po in pend_out:
            if po is not None:
                po.wait()

    return _sc_gather


@functools.cache
def _sc_cat_gather_fn():
    mesh = plsc.VectorSubcoreMesh(core_axis_name="c", subcore_axis_name="s")

    @functools.partial(
        pl.kernel,
        out_type=jax.ShapeDtypeStruct((B, 32), jnp.float32),
        mesh=mesh,
        compiler_params=pltpu.CompilerParams(use_tc_tiling_on_sc=False),
        scratch_types=[
            pltpu.VMEM((BPW,), jnp.int32),
            pltpu.VMEM((BPW, 32), jnp.float32),
            pltpu.SemaphoreType.DMA,
        ],
    )
    def _sc_cat(cat_tbl, cid, c_out, cidx_v, crows_v, sem):
        wid = lax.axis_index("s") * 2 + lax.axis_index("c")
        base = wid * BPW
        pltpu.sync_copy(cid.at[pl.ds(base, BPW)], cidx_v)
        pltpu.async_copy(cat_tbl.at[cidx_v], crows_v, sem).wait()
        pltpu.sync_copy(crows_v, c_out.at[pl.ds(base, BPW)])

    return _sc_cat


def _mlp_body(i_ref, c_ref, p_ref, wp_ref, bp_ref, w1a_ref, w1b_ref,
              w1c_ref, b1_ref, w2_ref, b2_ref, g_ref, bt_ref, o_ref):
    p = p_ref[...] * wp_ref[...] + bp_ref[...]
    h = jnp.dot(i_ref[...], w1a_ref[...], preferred_element_type=jnp.float32)
    h = h + jnp.dot(c_ref[...], w1b_ref[...], preferred_element_type=jnp.float32)
    h = h + jnp.dot(p, w1c_ref[...], preferred_element_type=jnp.float32)
    h = jnp.maximum(h + b1_ref[...], 0.0)
    o = jnp.dot(h, w2_ref[...], preferred_element_type=jnp.float32) + b2_ref[...]
    mu = jnp.mean(o, axis=-1, keepdims=True)
    d = o - mu
    var = jnp.mean(d * d, axis=-1, keepdims=True)
    o_ref[...] = d * lax.rsqrt(var + 1e-5) * g_ref[...] + bt_ref[...]


_MLP_BLOCK = 2048


def _mlp_call(i, c, p2, Wp, bp2, W1a, W1b, W1c, b12, W2, b22, g2, bt2):
    nblk = B // _MLP_BLOCK
    full = lambda shape: pl.BlockSpec(shape, lambda ib: (0, 0))
    return pl.pallas_call(
        _mlp_body,
        grid=(nblk,),
        in_specs=[
            pl.BlockSpec((_MLP_BLOCK, 64), lambda ib: (ib, 0)),
            pl.BlockSpec((_MLP_BLOCK, 32), lambda ib: (ib, 0)),
            pl.BlockSpec((_MLP_BLOCK, 1), lambda ib: (ib, 0)),
            full((1, 32)),
            full((1, 32)),
            full((64, 256)),
            full((32, 256)),
            full((32, 256)),
            full((1, 256)),
            full((256, 128)),
            full((1, 128)),
            full((1, 128)),
            full((1, 128)),
        ],
        out_specs=pl.BlockSpec((_MLP_BLOCK, 128), lambda ib: (ib, 0)),
        out_shape=jax.ShapeDtypeStruct((B, 128), jnp.float32),
    )(i, c, p2, Wp, bp2, W1a, W1b, W1c, b12, W2, b22, g2, bt2)


def kernel(item_id, category_id, price, item_table, cat_table,
           Wp, bp, W1, b1, W2, b2, gamma, beta):
    i = _sc_gather_fn()(item_table.reshape(125000, 8, 64),
                        item_id.astype(jnp.int32))
    c = _sc_cat_gather_fn()(cat_table, category_id.astype(jnp.int32))
    return _mlp_call(
        i, c, price[:, None], Wp, bp[None, :],
        W1[:64], W1[64:96], W1[96:128], b1[None, :],
        W2, b2[None, :], gamma[None, :], beta[None, :])


# speedup vs baseline: 1.0792x; 1.0082x over previous
import functools

import jax
import jax.numpy as jnp
from jax import lax
from jax.experimental import pallas as pl
from jax.experimental.pallas import tpu as pltpu
from jax.experimental.pallas import tpu_sc as plsc

B = 16384
NW = 32
BPW = B // NW
CHUNK = 32
NCHUNK = BPW // CHUNK


@functools.cache
def _sc_gather_fn():
    mesh = plsc.VectorSubcoreMesh(core_axis_name="c", subcore_axis_name="s")

    @functools.partial(
        pl.kernel,
        out_type=jax.ShapeDtypeStruct((B, 64), jnp.float32),
        mesh=mesh,
        compiler_params=pltpu.CompilerParams(needs_layout_passes=False),
        scratch_types=[
            pltpu.VMEM((BPW,), jnp.int32),
            pltpu.VMEM((CHUNK, 8, 64), jnp.float32),
            pltpu.VMEM((CHUNK, 8, 64), jnp.float32),
            pltpu.VMEM((CHUNK, 64), jnp.float32),
            pltpu.VMEM((CHUNK, 64), jnp.float32),
            pltpu.SemaphoreType.DMA,
            pltpu.SemaphoreType.DMA,
            pltpu.SemaphoreType.DMA,
        ],
    )
    def _sc_gather(tbl3, iid, i_out,
                   iidx_v, tile0, tile1, out0, out1, sem0, sem1, osem):
        wid = lax.axis_index("s") * 2 + lax.axis_index("c")
        base = wid * BPW
        pltpu.sync_copy(iid.at[pl.ds(base, BPW)], iidx_v)

        sems = (sem0, sem1)
        bufs = (tile0, tile1)
        pend = [None, None]
        iota = lax.iota(jnp.int32, 16)

        def fire(c):
            buf = bufs[c % 2]
            idparts = [iidx_v[pl.ds(c * CHUNK + 16 * q, 16)]
                       for q in range(CHUNK // 16)]
            cps = []
            for k in range(CHUNK):
                g = lax.div(idparts[k // 16][k % 16], 8)
                cps.append(pltpu.async_copy(
                    tbl3.at[pl.ds(g, 1)], buf.at[pl.ds(k, 1)],
                    sems[c % 2]))
            return cps

        obufs = (out0, out1)

        def extract(c):
            buf = bufs[c % 2]
            obuf = obufs[c % 2]

            def body(k, carry):
                j = c * CHUNK + k
                s16 = plsc.load_gather(iidx_v, [jnp.full((16,), j, jnp.int32)])
                r16 = lax.rem(s16, 8)
                k16 = jnp.full((16,), k, jnp.int32)
                for m in range(4):
                    vals = plsc.load_gather(buf, [k16, r16, iota + 16 * m])
                    obuf[k, pl.ds(16 * m, 16)] = vals
                return carry

            lax.fori_loop(0, CHUNK, body, 0)

        pend[0] = fire(0)
        pend_out = [None, None]
        for c in range(NCHUNK):
            if c + 1 < NCHUNK:
                pend[(c + 1) % 2] = fire(c + 1)
            for cp in pend[c % 2]:
                cp.wait()
            if pend_out[c % 2] is not None:
                pend_out[c % 2].wait()
            extract(c)
            pend_out[c % 2] = pltpu.async_copy(
                obufs[c % 2], i_out.at[pl.ds(base + c * CHUNK, CHUNK)], osem)
        for po in pend_out:
            if po is not None:
                po.wait()

    return _sc_gather


@functools.cache
def _sc_cat_gather_fn():
    mesh = plsc.VectorSubcoreMesh(core_axis_name="c", subcore_axis_name="s")

    @functools.partial(
        pl.kernel,
        out_type=jax.ShapeDtypeStruct((B, 32), jnp.float32),
        mesh=mesh,
        compiler_params=pltpu.CompilerParams(use_tc_tiling_on_sc=False),
        scratch_types=[
            pltpu.VMEM((BPW,), jnp.int32),
            pltpu.VMEM((BPW, 32), jnp.float32),
            pltpu.SemaphoreType.DMA,
        ],
    )
    def _sc_cat(cat_tbl, cid, c_out, cidx_v, crows_v, sem):
        wid = lax.axis_index("s") * 2 + lax.axis_index("c")
        base = wid * BPW
        pltpu.sync_copy(cid.at[pl.ds(base, BPW)], cidx_v)
        pltpu.async_copy(cat_tbl.at[cidx_v], crows_v, sem).wait()
        pltpu.sync_copy(crows_v, c_out.at[pl.ds(base, BPW)])

    return _sc_cat


def _mlp_body(i_ref, c_ref, p_ref, wp_ref, bp_ref, w1a_ref, w1b_ref,
              w1c_ref, b1_ref, w2_ref, b2_ref, g_ref, bt_ref, o_ref):
    p = p_ref[...] * wp_ref[...] + bp_ref[...]
    h = jnp.dot(i_ref[...], w1a_ref[...], preferred_element_type=jnp.float32)
    h = h + jnp.dot(c_ref[...], w1b_ref[...], preferred_element_type=jnp.float32)
    h = h + jnp.dot(p, w1c_ref[...], preferred_element_type=jnp.float32)
    h = jnp.maximum(h + b1_ref[...], 0.0)
    o = jnp.dot(h, w2_ref[...], preferred_element_type=jnp.float32) + b2_ref[...]
    mu = jnp.mean(o, axis=-1, keepdims=True)
    d = o - mu
    var = jnp.mean(d * d, axis=-1, keepdims=True)
    o_ref[...] = d * lax.rsqrt(var + 1e-5) * g_ref[...] + bt_ref[...]


_MLP_BLOCK = 4096


def _mlp_call(i, c, p2, Wp, bp2, W1a, W1b, W1c, b12, W2, b22, g2, bt2):
    nblk = B // _MLP_BLOCK
    full = lambda shape: pl.BlockSpec(shape, lambda ib: (0, 0))
    return pl.pallas_call(
        _mlp_body,
        grid=(nblk,),
        in_specs=[
            pl.BlockSpec((_MLP_BLOCK, 64), lambda ib: (ib, 0)),
            pl.BlockSpec((_MLP_BLOCK, 32), lambda ib: (ib, 0)),
            pl.BlockSpec((_MLP_BLOCK, 1), lambda ib: (ib, 0)),
            full((1, 32)),
            full((1, 32)),
            full((64, 256)),
            full((32, 256)),
            full((32, 256)),
            full((1, 256)),
            full((256, 128)),
            full((1, 128)),
            full((1, 128)),
            full((1, 128)),
        ],
        out_specs=pl.BlockSpec((_MLP_BLOCK, 128), lambda ib: (ib, 0)),
        out_shape=jax.ShapeDtypeStruct((B, 128), jnp.float32),
    )(i, c, p2, Wp, bp2, W1a, W1b, W1c, b12, W2, b22, g2, bt2)


def kernel(item_id, category_id, price, item_table, cat_table,
           Wp, bp, W1, b1, W2, b2, gamma, beta):
    i = _sc_gather_fn()(item_table.reshape(125000, 8, 64),
                        item_id.astype(jnp.int32))
    c = _sc_cat_gather_fn()(cat_table, category_id.astype(jnp.int32))
    return _mlp_call(
        i, c, price[:, None], Wp, bp[None, :],
        W1[:64], W1[64:96], W1[96:128], b1[None, :],
        W2, b2[None, :], gamma[None, :], beta[None, :])
